# Initial kernel scaffold; baseline (speedup 1.0000x reference)
import jax
import jax.numpy as jnp
from jax.experimental import pallas as pl


def kernel(conditions, tables):
    raise NotImplementedError("write your pallas kernel here")


# speedup vs baseline: 1.1421x; 1.1421x over previous
import jax
import jax.numpy as jnp
from jax import lax
from jax.experimental import pallas as pl
from jax.experimental.pallas import tpu as pltpu
from jax.experimental.pallas import tpu_sc as plsc

N_DOMAIN = 26
VOCAB = 100000
D_EMB = 32
BATCH = 16384

R = BATCH * N_DOMAIN
NW = 32
RW = R // NW
CHUNK = 1024
NCH = RW // CHUNK
IDX_ROWS = CHUNK // 128
LANES = 16


def _sc_body(cond_hbm, table_hbm, out_hbm, cond_v, idx_v, rows_v, sem):
    wid = lax.axis_index("s") * 2 + lax.axis_index("c")
    base = wid * RW
    iota = lax.iota(jnp.int32, LANES)

    def chunk_body(g, carry):
        start = base + g * CHUNK
        pltpu.sync_copy(cond_hbm.at[pl.ds(start, CHUNK)], cond_v)
        for r in range(IDX_ROWS):
            for u in range(128 // LANES):
                pos = r * 128 + u * LANES
                lanes = (start + pos) + iota
                offs = lax.rem(lanes, N_DOMAIN) * VOCAB
                idx_v[r, pl.ds(u * LANES, LANES)] = (
                    cond_v[pl.ds(pos, LANES)] + offs
                )
        copies = [
            pltpu.async_copy(
                table_hbm.at[idx_v.at[r]],
                rows_v.at[pl.ds(r * 128, 128)],
                sem,
            )
            for r in range(IDX_ROWS)
        ]
        for cp in copies:
            cp.wait()
        pltpu.sync_copy(rows_v, out_hbm.at[pl.ds(start, CHUNK)])
        return carry

    lax.fori_loop(0, NCH, chunk_body, 0)


@jax.jit
def _sc_gather(cond_flat, table_flat):
    mesh = plsc.VectorSubcoreMesh(core_axis_name="c", subcore_axis_name="s")
    return pl.kernel(
        _sc_body,
        out_type=jax.ShapeDtypeStruct((R, D_EMB), jnp.float32),
        mesh=mesh,
        scratch_types=[
            pltpu.VMEM((CHUNK,), jnp.int32),
            pltpu.VMEM((IDX_ROWS, 128), jnp.int32),
            pltpu.VMEM((CHUNK, D_EMB), jnp.float32),
            pltpu.SemaphoreType.DMA,
        ],
        compiler_params=pltpu.CompilerParams(use_tc_tiling_on_sc=False),
    )(cond_flat, table_flat)


def kernel(conditions, tables):
    cond_flat = conditions.reshape(R)
    table_flat = tables.reshape(N_DOMAIN * VOCAB, D_EMB)
    out = _sc_gather(cond_flat, table_flat)
    return out.reshape(BATCH, N_DOMAIN, D_EMB)


# speedup vs baseline: 3.8535x; 3.3740x over previous
import jax
import jax.numpy as jnp
from jax import lax
from jax.experimental import pallas as pl
from jax.experimental.pallas import tpu as pltpu
from jax.experimental.pallas import tpu_sc as plsc

N_DOMAIN = 26
VOCAB = 100000
D_EMB = 32
BATCH = 16384

NROWS = N_DOMAIN * D_EMB
NW = 32
LANES = 16
BCH = 4096
NBCH = BATCH // BCH


def _sc_body(cond_hbm, table_hbm, out_hbm, row_v, cond_v, out_v, sem):
    e = lax.axis_index("s") * 2 + lax.axis_index("c")

    def domain_body(i, carry):
        r = i * D_EMB + e
        pltpu.sync_copy(table_hbm.at[r], row_v)
        pltpu.sync_copy(cond_hbm.at[i], cond_v)

        def chunk_body(c, carry2):
            base = c * BCH
            for t in range(BCH // LANES):
                idx = cond_v[pl.ds(base + t * LANES, LANES)]
                out_v[pl.ds(t * LANES, LANES)] = plsc.load_gather(
                    row_v, [idx]
                )
            pltpu.sync_copy(out_v, out_hbm.at[r, pl.ds(base, BCH)])
            return carry2

        lax.fori_loop(0, NBCH, chunk_body, 0)
        return carry

    lax.fori_loop(0, N_DOMAIN, domain_body, 0)


@jax.jit
def _sc_gather(cond_t, table_t):
    mesh = plsc.VectorSubcoreMesh(core_axis_name="c", subcore_axis_name="s")
    return pl.kernel(
        _sc_body,
        out_type=jax.ShapeDtypeStruct((NROWS, BATCH), jnp.float32),
        mesh=mesh,
        scratch_types=[
            pltpu.VMEM((VOCAB,), jnp.float32),
            pltpu.VMEM((BATCH,), jnp.int32),
            pltpu.VMEM((BCH,), jnp.float32),
            pltpu.SemaphoreType.DMA,
        ],
        compiler_params=pltpu.CompilerParams(
            use_tc_tiling_on_sc=True, needs_layout_passes=False
        ),
    )(cond_t, table_t)


def kernel(conditions, tables):
    cond_t = conditions.T
    table_t = tables.transpose(0, 2, 1).reshape(NROWS, VOCAB)
    out = _sc_gather(cond_t, table_t)
    return out.reshape(N_DOMAIN, D_EMB, BATCH).transpose(2, 0, 1)


# speedup vs baseline: 6.7514x; 1.7520x over previous
import jax
import jax.numpy as jnp
from jax import lax
from jax.experimental import pallas as pl
from jax.experimental.pallas import tpu as pltpu
from jax.experimental.pallas import tpu_sc as plsc

N_DOMAIN = 26
VOCAB = 100000
D_EMB = 32
BATCH = 16384

NROWS = N_DOMAIN * D_EMB
NW = 32
LANES = 16
BCH = 4096
NBCH = BATCH // BCH


def _sc_body(cond_hbm, table_hbm, out_hbm, row_v, cond_v, out_v, sem_in, sem_out):
    e = lax.axis_index("s") * 2 + lax.axis_index("c")
    G = 8

    def gather_chunk(base, buf):
        for t0 in range(0, BCH // LANES, G):
            idxs = [
                cond_v[pl.ds(base + (t0 + u) * LANES, LANES)] for u in range(G)
            ]
            vals = [plsc.load_gather(row_v, [idxs[u]]) for u in range(G)]
            for u in range(G):
                out_v[buf, pl.ds((t0 + u) * LANES, LANES)] = vals[u]

    QBOUNDS = [0, 25088, 50176, 75264, VOCAB]

    def drain_out(r):
        pltpu.make_async_copy(
            out_v.at[0], out_hbm.at[r, pl.ds(0, BCH)], sem_out
        ).wait()
        pltpu.make_async_copy(
            out_v.at[1], out_hbm.at[r, pl.ds(0, BCH)], sem_out
        ).wait()

    def domain_body(i, carry):
        r = i * D_EMB + e
        row_cps = [pltpu.async_copy(table_hbm.at[r], row_v, sem_in)]
        cp_cond = pltpu.async_copy(cond_hbm.at[i], cond_v, sem_in)

        @pl.when(i > 0)
        def _():
            drain_out(r)

        for cp in row_cps:
            cp.wait()
        cp_cond.wait()
        for c in range(NBCH):
            buf = c % 2
            if c >= 2:
                pltpu.make_async_copy(
                    out_v.at[buf], out_hbm.at[r, pl.ds(0, BCH)], sem_out
                ).wait()
            gather_chunk(c * BCH, buf)
            pltpu.async_copy(
                out_v.at[buf], out_hbm.at[r, pl.ds(c * BCH, BCH)], sem_out
            )
        return carry

    lax.fori_loop(0, N_DOMAIN, domain_body, 0)
    drain_out(0)


@jax.jit
def _sc_gather(cond_t, table_t):
    mesh = plsc.VectorSubcoreMesh(core_axis_name="c", subcore_axis_name="s")
    return pl.kernel(
        _sc_body,
        out_type=jax.ShapeDtypeStruct((NROWS, BATCH), jnp.float32),
        mesh=mesh,
        scratch_types=[
            pltpu.VMEM((VOCAB,), jnp.float32),
            pltpu.VMEM((BATCH,), jnp.int32),
            pltpu.VMEM((2, BCH), jnp.float32),
            pltpu.SemaphoreType.DMA,
            pltpu.SemaphoreType.DMA,
        ],
        compiler_params=pltpu.CompilerParams(
            use_tc_tiling_on_sc=True, needs_layout_passes=False
        ),
    )(cond_t, table_t)


def kernel(conditions, tables):
    cond_t = conditions.T
    table_t = tables.transpose(0, 2, 1).reshape(NROWS, VOCAB)
    out = _sc_gather(cond_t, table_t)
    return out.reshape(N_DOMAIN, D_EMB, BATCH).transpose(2, 0, 1)
